# baseline (speedup 1.0000x reference)
import functools

import jax
import jax.numpy as jnp
from jax import lax
from jax.experimental import pallas as pl
from jax.experimental.pallas import tpu as pltpu
from jax.experimental.pallas import tpu_sc as plsc

B, T, L, M, S, N, D = 8, 5, 77, 5, 50, 2048, 512
K = 50
NC, NS = 2, 16
NW = NC * NS
TOKP = 3328
TOK_PER_W = TOKP // NW
CANDP = 512
CAND_PER_W = CANDP // NW
PRE_PER_W = 16


def _score_topk_body(txtm_ref, act_ref, w0_ref, b0_ref, pdb_ref, tk_ref):
    x = txtm_ref[...]
    s = jnp.einsum('bqd,bkd->bqk', x, x,
                   preferred_element_type=jnp.float32) / jnp.sqrt(
                       jnp.float32(D))
    s = s - jnp.max(s, axis=-1, keepdims=True)
    e = jnp.exp(s)
    w = e / jnp.sum(e, axis=-1, keepdims=True)
    a = jnp.einsum('bqk,bkd->bqd', w, x,
                   preferred_element_type=jnp.float32)
    t = jnp.tanh(jnp.einsum('bqd,dj->bqj', a, w0_ref[...],
                            preferred_element_type=jnp.float32)
                 + b0_ref[0, 0])
    wt = t[..., 0]
    sc = jnp.einsum('bq,bqn->bn', wt, act_ref[...],
                    preferred_element_type=jnp.float32)
    mu = jnp.mean(sc, axis=-1, keepdims=True)
    var = jnp.sum((sc - mu) ** 2, axis=-1, keepdims=True) / jnp.float32(N - 1)
    z = (sc - mu) / jnp.sqrt(var)
    pdb = 1.0 / (1.0 + jnp.exp(-z))
    pdb = jnp.clip(pdb, 1e-10, 1.0 - 1e-10)
    pdb_ref[...] = pdb

    iota = lax.broadcasted_iota(jnp.int32, (B, N), 1)
    vals = pdb
    cols = []
    for j in range(K):
        m = jnp.max(vals, axis=-1, keepdims=True)
        am = jnp.min(jnp.where(vals == m, iota, N), axis=-1)
        cols.append(am[:, None])
        vals = jnp.where(iota == am[:, None], -1.0, vals)
    tk2d = jnp.concatenate(cols, axis=1).astype(jnp.int32)
    tk_ref[...] = tk2d


def _score_topk(txt_matric, actions_matric, w0, b0):
    return pl.pallas_call(
        _score_topk_body,
        out_shape=[
            jax.ShapeDtypeStruct((B, N), jnp.float32),
            jax.ShapeDtypeStruct((B, K), jnp.int32),
        ],
    )(txt_matric, actions_matric, w0, b0)


def _sc_gather_body(tok_tab, tok_idx, emb_tab, pre_tab, tk_hbm,
                    tok_out, cand_out, pre_out,
                    tokidx_v, tokbuf_v, tkp_v, cbuf_v, sem, psem):
    wid = lax.axis_index("s") * NC + lax.axis_index("c")

    tbase = wid * TOK_PER_W
    pltpu.sync_copy(tok_idx.at[pl.ds(tbase, TOK_PER_W)], tokidx_v)
    pltpu.async_copy(tok_tab.at[tokidx_v], tokbuf_v, sem).wait()
    pltpu.sync_copy(tokbuf_v, tok_out.at[pl.ds(tbase, TOK_PER_W)])

    pltpu.sync_copy(tk_hbm, tkp_v)

    cbase = wid * CAND_PER_W
    pltpu.async_copy(emb_tab.at[tkp_v.at[pl.ds(cbase, CAND_PER_W)]],
                     cbuf_v, sem).wait()
    pltpu.sync_copy(cbuf_v, cand_out.at[pl.ds(cbase, CAND_PER_W)])

    pbase = wid * PRE_PER_W
    srcv = tkp_v[pl.ds(pbase, PRE_PER_W)]
    for j in range(PRE_PER_W):
        item = pbase + j

        @pl.when(item < B * K)
        def _():
            pltpu.async_copy(pre_tab.at[pl.ds(srcv[j], 1)],
                             pre_out.at[pl.ds(item, 1)], psem)
    for j in range(PRE_PER_W):
        item = pbase + j

        @pl.when(item < B * K)
        def _():
            pltpu.make_async_copy(pre_tab.at[pl.ds(srcv[j], 1)],
                                  pre_out.at[pl.ds(item, 1)], psem).wait()


@functools.partial(jax.jit, static_argnums=())
def _sc_gather(tok_embed, tok_idx, data_emb, pre4, tk512):
    mesh = plsc.VectorSubcoreMesh(core_axis_name="c", subcore_axis_name="s",
                                  num_cores=NC, num_subcores=NS)
    kfn = pl.kernel(
        _sc_gather_body,
        out_type=[
            jax.ShapeDtypeStruct((TOKP, D), jnp.float32),
            jax.ShapeDtypeStruct((CANDP, D), jnp.float32),
            jax.ShapeDtypeStruct((B * K, 3, 224, 224), jnp.float32),
        ],
        mesh=mesh,
        scratch_types=[
            pltpu.VMEM((TOK_PER_W,), jnp.int32),
            pltpu.VMEM((TOK_PER_W, D), jnp.float32),
            pltpu.VMEM((CANDP,), jnp.int32),
            pltpu.VMEM((CAND_PER_W, D), jnp.float32),
            pltpu.SemaphoreType.DMA,
            pltpu.SemaphoreType.DMA,
        ],
    )
    return kfn(tok_embed, tok_idx, data_emb, pre4, tk512)


def _heads_body(tok_ref, img_ref, cand_ref, tkb_ref, did_ref, wp_ref, bp_ref,
                wq_ref, bq_ref, p_ref, q_ref, mc_ref, mid_ref, cid_ref):
    iota_n = lax.broadcasted_iota(jnp.int32, (B, K, N), 2)
    oh_n = iota_n == tkb_ref[...][:, :, None]
    cid = jnp.sum(jnp.where(oh_n, did_ref[...][0][None, None, :], 0.0),
                  axis=-1)
    cid_ref[...] = cid
    rows = tok_ref[...][:B * T * L]
    te = jnp.mean(rows.reshape(B, T, L, D), axis=2)
    te = te / jnp.sqrt(jnp.sum(te * te, axis=-1, keepdims=True))
    fusion = jnp.concatenate([te, img_ref[...]], axis=1)

    cand = cand_ref[...]
    s = jnp.einsum('bkd,bfd->bkf', cand, fusion,
                   preferred_element_type=jnp.float32) / jnp.sqrt(
                       jnp.float32(D))
    s = s - jnp.max(s, axis=-1, keepdims=True)
    e = jnp.exp(s)
    w = e / jnp.sum(e, axis=-1, keepdims=True)
    xc = jnp.einsum('bkf,bfd->bkd', w, fusion,
                    preferred_element_type=jnp.float32)

    sp = jnp.einsum('bkd,dj->bkj', xc, wp_ref[...],
                    preferred_element_type=jnp.float32)[..., 0] + bp_ref[0, 0]
    sp = sp - jnp.max(sp, axis=-1, keepdims=True)
    ep = jnp.exp(sp)
    p = ep / jnp.sum(ep, axis=-1, keepdims=True)
    p = jnp.clip(p, 1e-10, 1.0 - 1e-10)
    q = jnp.tanh(jnp.einsum('bkd,dj->bkj', xc, wq_ref[...],
                            preferred_element_type=jnp.float32)[..., 0]
                 + bq_ref[0, 0])

    p_ref[...] = p
    q_ref[...] = q

    iota = lax.broadcasted_iota(jnp.int32, (B, K), 1)
    m = jnp.max(p, axis=-1, keepdims=True)
    am = jnp.min(jnp.where(p == m, iota, K), axis=-1)
    oh = (iota == am[:, None]).astype(jnp.float32)
    mc_ref[...] = jnp.einsum('bk,bkd->bd', oh, cand,
                             preferred_element_type=jnp.float32)
    mid_ref[...] = jnp.sum(oh * cid, axis=1, keepdims=True)


def _heads(tok_rows, img, candidates, tkb, did, wp, bp, wq, bq):
    return pl.pallas_call(
        _heads_body,
        out_shape=[
            jax.ShapeDtypeStruct((B, K), jnp.float32),
            jax.ShapeDtypeStruct((B, K), jnp.float32),
            jax.ShapeDtypeStruct((B, D), jnp.float32),
            jax.ShapeDtypeStruct((B, 1), jnp.float32),
            jax.ShapeDtypeStruct((B, K), jnp.float32),
        ],
    )(tok_rows, img, candidates, tkb, did, wp, bp, wq, bq)


def kernel(img, txt_tokens, txt_matric, actions_matric, data_emb, data_pre,
           data_id, tok_embed, W_seq0, b_seq0, W_p, b_p, W_q, b_q, k):
    del k
    b0 = b_seq0.reshape(1, 1).astype(jnp.float32)
    bp = b_p.reshape(1, 1).astype(jnp.float32)
    bq = b_q.reshape(1, 1).astype(jnp.float32)

    p_db, tk2d = _score_topk(txt_matric, actions_matric, W_seq0, b0)

    tokflat = txt_tokens.reshape(B * T * L).astype(jnp.int32)
    tok_idx = jnp.concatenate(
        [tokflat, jnp.zeros((TOKP - B * T * L,), jnp.int32)])
    tk512 = jnp.concatenate(
        [tk2d.reshape(B * K), jnp.zeros((CANDP - B * K,), jnp.int32)])
    tok_rows, cand_b, pre_out = _sc_gather(
        tok_embed, tok_idx, data_emb, data_pre, tk512)

    candidates = cand_b[:B * K].reshape(B, K, D)
    candidates_pre = pre_out.reshape(B, K, 3, 224, 224)

    p, q, max_candidates, mid, candidates_id = _heads(
        tok_rows, img, candidates, tk2d, data_id.reshape(1, N),
        W_p, bp, W_q, bq)
    max_candidates_id = mid.reshape(B)

    return (p, q, max_candidates, max_candidates_id, candidates,
            candidates_id, candidates_pre, p_db)


# speedup vs baseline: 6.1414x; 6.1414x over previous
import functools

import jax
import jax.numpy as jnp
from jax import lax
from jax.experimental import pallas as pl
from jax.experimental.pallas import tpu as pltpu
from jax.experimental.pallas import tpu_sc as plsc

B, T, L, M, S, N, D = 8, 5, 77, 5, 50, 2048, 512
K = 50
NC, NS = 2, 16
NW = NC * NS
TOKP = 3328
TOK_PER_W = TOKP // NW
CANDP = 512
CAND_PER_W = CANDP // NW
PRE_PER_W = 16


def _score_topk_body(txtm_ref, act_ref, w0_ref, b0_ref, pdb_ref, tk_ref):
    x = txtm_ref[...]
    s = jnp.einsum('bqd,bkd->bqk', x, x,
                   preferred_element_type=jnp.float32) / jnp.sqrt(
                       jnp.float32(D))
    s = s - jnp.max(s, axis=-1, keepdims=True)
    e = jnp.exp(s)
    w = e / jnp.sum(e, axis=-1, keepdims=True)
    a = jnp.einsum('bqk,bkd->bqd', w, x,
                   preferred_element_type=jnp.float32)
    t = jnp.tanh(jnp.einsum('bqd,dj->bqj', a, w0_ref[...],
                            preferred_element_type=jnp.float32)
                 + b0_ref[0, 0])
    wt = t[..., 0]
    sc = jnp.einsum('bq,bqn->bn', wt, act_ref[...],
                    preferred_element_type=jnp.float32)
    mu = jnp.mean(sc, axis=-1, keepdims=True)
    var = jnp.sum((sc - mu) ** 2, axis=-1, keepdims=True) / jnp.float32(N - 1)
    z = (sc - mu) / jnp.sqrt(var)
    pdb = 1.0 / (1.0 + jnp.exp(-z))
    pdb = jnp.clip(pdb, 1e-10, 1.0 - 1e-10)
    pdb_ref[...] = pdb

    iota = lax.broadcasted_iota(jnp.int32, (B, N), 1)
    vals = pdb
    cols = []
    for j in range(K):
        m = jnp.max(vals, axis=-1, keepdims=True)
        am = jnp.min(jnp.where(vals == m, iota, N), axis=-1)
        cols.append(am[:, None])
        vals = jnp.where(iota == am[:, None], -1.0, vals)
    tk2d = jnp.concatenate(cols, axis=1).astype(jnp.int32)
    tk_ref[...] = tk2d


def _score_topk(txt_matric, actions_matric, w0, b0):
    return pl.pallas_call(
        _score_topk_body,
        out_shape=[
            jax.ShapeDtypeStruct((B, N), jnp.float32),
            jax.ShapeDtypeStruct((B, K), jnp.int32),
        ],
    )(txt_matric, actions_matric, w0, b0)


def _sc_gather_body(tok_tab, tok_idx, emb_tab, pre_tab, tk_hbm,
                    tok_out, cand_out, pre_out,
                    tokidx_v, tokbuf_v, tkp_v, cbuf_v, pbuf_v, sem, psem):
    wid = lax.axis_index("s") * NC + lax.axis_index("c")

    tbase = wid * TOK_PER_W
    pltpu.sync_copy(tok_idx.at[pl.ds(tbase, TOK_PER_W)], tokidx_v)
    pltpu.async_copy(tok_tab.at[tokidx_v], tokbuf_v, sem).wait()
    pltpu.sync_copy(tokbuf_v, tok_out.at[pl.ds(tbase, TOK_PER_W)])

    pltpu.sync_copy(tk_hbm, tkp_v)

    cbase = wid * CAND_PER_W
    pltpu.async_copy(emb_tab.at[tkp_v.at[pl.ds(cbase, CAND_PER_W)]],
                     cbuf_v, sem).wait()
    pltpu.sync_copy(cbuf_v, cand_out.at[pl.ds(cbase, CAND_PER_W)])

    pbase = wid * PRE_PER_W
    srcv = tkp_v[pl.ds(pbase, PRE_PER_W)]
    for j in range(PRE_PER_W):
        item = pbase + j

        @pl.when(item < B * K)
        def _():
            for c in range(3):
                pltpu.async_copy(
                    pre_tab.at[pl.ds(srcv[j], 1), pl.ds(c, 1)],
                    pbuf_v, psem).wait()
                pltpu.sync_copy(
                    pbuf_v, pre_out.at[pl.ds(item, 1), pl.ds(c, 1)])


@functools.partial(jax.jit, static_argnums=())
def _sc_gather(tok_embed, tok_idx, data_emb, pre4, tk512):
    mesh = plsc.VectorSubcoreMesh(core_axis_name="c", subcore_axis_name="s",
                                  num_cores=NC, num_subcores=NS)
    kfn = pl.kernel(
        _sc_gather_body,
        out_type=[
            jax.ShapeDtypeStruct((TOKP, D), jnp.float32),
            jax.ShapeDtypeStruct((CANDP, D), jnp.float32),
            jax.ShapeDtypeStruct((B * K, 3, 224, 224), jnp.float32),
        ],
        mesh=mesh,
        scratch_types=[
            pltpu.VMEM((TOK_PER_W,), jnp.int32),
            pltpu.VMEM((TOK_PER_W, D), jnp.float32),
            pltpu.VMEM((CANDP,), jnp.int32),
            pltpu.VMEM((CAND_PER_W, D), jnp.float32),
            pltpu.VMEM((1, 1, 224, 224), jnp.float32),
            pltpu.SemaphoreType.DMA,
            pltpu.SemaphoreType.DMA,
        ],
    )
    return kfn(tok_embed, tok_idx, data_emb, pre4, tk512)


def _heads_body(tok_ref, img_ref, cand_ref, tkb_ref, did_ref, wp_ref, bp_ref,
                wq_ref, bq_ref, p_ref, q_ref, mc_ref, mid_ref, cid_ref):
    iota_n = lax.broadcasted_iota(jnp.int32, (B, K, N), 2)
    oh_n = iota_n == tkb_ref[...][:, :, None]
    cid = jnp.sum(jnp.where(oh_n, did_ref[...][0][None, None, :], 0.0),
                  axis=-1)
    cid_ref[...] = cid
    rows = tok_ref[...][:B * T * L]
    te = jnp.mean(rows.reshape(B, T, L, D), axis=2)
    te = te / jnp.sqrt(jnp.sum(te * te, axis=-1, keepdims=True))
    fusion = jnp.concatenate([te, img_ref[...]], axis=1)

    cand = cand_ref[...]
    s = jnp.einsum('bkd,bfd->bkf', cand, fusion,
                   preferred_element_type=jnp.float32) / jnp.sqrt(
                       jnp.float32(D))
    s = s - jnp.max(s, axis=-1, keepdims=True)
    e = jnp.exp(s)
    w = e / jnp.sum(e, axis=-1, keepdims=True)
    xc = jnp.einsum('bkf,bfd->bkd', w, fusion,
                    preferred_element_type=jnp.float32)

    sp = jnp.einsum('bkd,dj->bkj', xc, wp_ref[...],
                    preferred_element_type=jnp.float32)[..., 0] + bp_ref[0, 0]
    sp = sp - jnp.max(sp, axis=-1, keepdims=True)
    ep = jnp.exp(sp)
    p = ep / jnp.sum(ep, axis=-1, keepdims=True)
    p = jnp.clip(p, 1e-10, 1.0 - 1e-10)
    q = jnp.tanh(jnp.einsum('bkd,dj->bkj', xc, wq_ref[...],
                            preferred_element_type=jnp.float32)[..., 0]
                 + bq_ref[0, 0])

    p_ref[...] = p
    q_ref[...] = q

    iota = lax.broadcasted_iota(jnp.int32, (B, K), 1)
    m = jnp.max(p, axis=-1, keepdims=True)
    am = jnp.min(jnp.where(p == m, iota, K), axis=-1)
    oh = (iota == am[:, None]).astype(jnp.float32)
    mc_ref[...] = jnp.einsum('bk,bkd->bd', oh, cand,
                             preferred_element_type=jnp.float32)
    mid_ref[...] = jnp.sum(oh * cid, axis=1, keepdims=True)


def _heads(tok_rows, img, candidates, tkb, did, wp, bp, wq, bq):
    return pl.pallas_call(
        _heads_body,
        out_shape=[
            jax.ShapeDtypeStruct((B, K), jnp.float32),
            jax.ShapeDtypeStruct((B, K), jnp.float32),
            jax.ShapeDtypeStruct((B, D), jnp.float32),
            jax.ShapeDtypeStruct((B, 1), jnp.float32),
            jax.ShapeDtypeStruct((B, K), jnp.float32),
        ],
    )(tok_rows, img, candidates, tkb, did, wp, bp, wq, bq)


def kernel(img, txt_tokens, txt_matric, actions_matric, data_emb, data_pre,
           data_id, tok_embed, W_seq0, b_seq0, W_p, b_p, W_q, b_q, k):
    del k
    b0 = b_seq0.reshape(1, 1).astype(jnp.float32)
    bp = b_p.reshape(1, 1).astype(jnp.float32)
    bq = b_q.reshape(1, 1).astype(jnp.float32)

    p_db, tk2d = _score_topk(txt_matric, actions_matric, W_seq0, b0)

    tokflat = txt_tokens.reshape(B * T * L).astype(jnp.int32)
    tok_idx = jnp.concatenate(
        [tokflat, jnp.zeros((TOKP - B * T * L,), jnp.int32)])
    tk512 = jnp.concatenate(
        [tk2d.reshape(B * K), jnp.zeros((CANDP - B * K,), jnp.int32)])
    tok_rows, cand_b, pre_out = _sc_gather(
        tok_embed, tok_idx, data_emb, data_pre, tk512)

    candidates = cand_b[:B * K].reshape(B, K, D)
    candidates_pre = pre_out.reshape(B, K, 3, 224, 224)

    p, q, max_candidates, mid, candidates_id = _heads(
        tok_rows, img, candidates, tk2d, data_id.reshape(1, N),
        W_p, bp, W_q, bq)
    max_candidates_id = mid.reshape(B)

    return (p, q, max_candidates, max_candidates_id, candidates,
            candidates_id, candidates_pre, p_db)


# speedup vs baseline: 17.6970x; 2.8816x over previous
import jax
import jax.numpy as jnp
from jax import lax
from jax.experimental import pallas as pl
from jax.experimental.pallas import tpu as pltpu
from jax.experimental.pallas import tpu_sc as plsc

B, T, L, M, S, N, D = 8, 5, 77, 5, 50, 2048, 512
K = 50
NC, NS = 2, 16
NW = NC * NS
TOKP = 3328
TOK_PER_W = TOKP // NW
CANDP = 512
CAND_PER_W = CANDP // NW


def _score_topk_body(txtm_ref, act_ref, w0_ref, b0_ref, pdb_ref, tk_ref):
    x = txtm_ref[...]
    s = jnp.einsum('bqd,bkd->bqk', x, x,
                   preferred_element_type=jnp.float32) / jnp.sqrt(
                       jnp.float32(D))
    s = s - jnp.max(s, axis=-1, keepdims=True)
    e = jnp.exp(s)
    w = e / jnp.sum(e, axis=-1, keepdims=True)
    a = jnp.einsum('bqk,bkd->bqd', w, x,
                   preferred_element_type=jnp.float32)
    t = jnp.tanh(jnp.einsum('bqd,dj->bqj', a, w0_ref[...],
                            preferred_element_type=jnp.float32)
                 + b0_ref[0, 0])
    wt = t[..., 0]
    sc = jnp.einsum('bq,bqn->bn', wt, act_ref[...],
                    preferred_element_type=jnp.float32)
    mu = jnp.mean(sc, axis=-1, keepdims=True)
    var = jnp.sum((sc - mu) ** 2, axis=-1, keepdims=True) / jnp.float32(N - 1)
    z = (sc - mu) / jnp.sqrt(var)
    pdb = 1.0 / (1.0 + jnp.exp(-z))
    pdb = jnp.clip(pdb, 1e-10, 1.0 - 1e-10)
    pdb_ref[...] = pdb

    iota = lax.broadcasted_iota(jnp.int32, (B, N), 1)
    vals = pdb
    cols = []
    for j in range(K):
        m = jnp.max(vals, axis=-1, keepdims=True)
        am = jnp.min(jnp.where(vals == m, iota, N), axis=-1)
        cols.append(am[:, None])
        vals = jnp.where(iota == am[:, None], -1.0, vals)
    tk2d = jnp.concatenate(cols, axis=1).astype(jnp.int32)
    tk_ref[...] = tk2d


def _score_topk(txt_matric, actions_matric, w0, b0):
    return pl.pallas_call(
        _score_topk_body,
        out_shape=[
            jax.ShapeDtypeStruct((B, N), jnp.float32),
            jax.ShapeDtypeStruct((B, K), jnp.int32),
        ],
    )(txt_matric, actions_matric, w0, b0)


def _sc_gather_body(tok_tab, tok_idx, emb_tab, tk_hbm,
                    tok_out, cand_out,
                    tokidx_v, tokbuf_v, tkp_v, cbuf_v, sem):
    wid = lax.axis_index("s") * NC + lax.axis_index("c")

    tbase = wid * TOK_PER_W
    pltpu.sync_copy(tok_idx.at[pl.ds(tbase, TOK_PER_W)], tokidx_v)
    pltpu.async_copy(tok_tab.at[tokidx_v], tokbuf_v, sem).wait()
    pltpu.sync_copy(tokbuf_v, tok_out.at[pl.ds(tbase, TOK_PER_W)])

    pltpu.sync_copy(tk_hbm, tkp_v)

    cbase = wid * CAND_PER_W
    pltpu.async_copy(emb_tab.at[tkp_v.at[pl.ds(cbase, CAND_PER_W)]],
                     cbuf_v, sem).wait()
    pltpu.sync_copy(cbuf_v, cand_out.at[pl.ds(cbase, CAND_PER_W)])


def _sc_gather(tok_embed, tok_idx, data_emb, tk512):
    mesh = plsc.VectorSubcoreMesh(core_axis_name="c", subcore_axis_name="s",
                                  num_cores=NC, num_subcores=NS)
    kfn = pl.kernel(
        _sc_gather_body,
        out_type=[
            jax.ShapeDtypeStruct((TOKP, D), jnp.float32),
            jax.ShapeDtypeStruct((CANDP, D), jnp.float32),
        ],
        mesh=mesh,
        scratch_types=[
            pltpu.VMEM((TOK_PER_W,), jnp.int32),
            pltpu.VMEM((TOK_PER_W, D), jnp.float32),
            pltpu.VMEM((CANDP,), jnp.int32),
            pltpu.VMEM((CAND_PER_W, D), jnp.float32),
            pltpu.SemaphoreType.DMA,
        ],
    )
    return kfn(tok_embed, tok_idx, data_emb, tk512)


YB = 8


def _pre_gather_body(oh_ref, blk_ref, out_ref):
    oh = oh_ref[...]
    yb = blk_ref[...].reshape(YB * 224, N).astype(jnp.bfloat16)
    r = lax.dot_general(oh, yb, (((1,), (1,)), ((), ())),
                        preferred_element_type=jnp.float32)
    out_ref[...] = r.reshape(B * K, 1, YB, 224)


def _pre_gather(onehot, pre_t):
    return pl.pallas_call(
        _pre_gather_body,
        grid=(3, 224 // YB),
        in_specs=[
            pl.BlockSpec((B * K, N), lambda c, y: (0, 0)),
            pl.BlockSpec((1, YB, 224, N), lambda c, y: (c, y, 0, 0)),
        ],
        out_specs=pl.BlockSpec((B * K, 1, YB, 224), lambda c, y: (0, c, y, 0)),
        out_shape=jax.ShapeDtypeStruct((B * K, 3, 224, 224), jnp.float32),
    )(onehot, pre_t)


def _heads_body(tok_ref, img_ref, cand_ref, tkb_ref, did_ref, wp_ref, bp_ref,
                wq_ref, bq_ref, p_ref, q_ref, mc_ref, mid_ref, cid_ref):
    iota_n = lax.broadcasted_iota(jnp.int32, (B, K, N), 2)
    oh_n = iota_n == tkb_ref[...][:, :, None]
    cid = jnp.sum(jnp.where(oh_n, did_ref[...][0][None, None, :], 0.0),
                  axis=-1)
    cid_ref[...] = cid
    rows = tok_ref[...][:B * T * L]
    te = jnp.mean(rows.reshape(B, T, L, D), axis=2)
    te = te / jnp.sqrt(jnp.sum(te * te, axis=-1, keepdims=True))
    fusion = jnp.concatenate([te, img_ref[...]], axis=1)

    cand = cand_ref[...]
    s = jnp.einsum('bkd,bfd->bkf', cand, fusion,
                   preferred_element_type=jnp.float32) / jnp.sqrt(
                       jnp.float32(D))
    s = s - jnp.max(s, axis=-1, keepdims=True)
    e = jnp.exp(s)
    w = e / jnp.sum(e, axis=-1, keepdims=True)
    xc = jnp.einsum('bkf,bfd->bkd', w, fusion,
                    preferred_element_type=jnp.float32)

    sp = jnp.einsum('bkd,dj->bkj', xc, wp_ref[...],
                    preferred_element_type=jnp.float32)[..., 0] + bp_ref[0, 0]
    sp = sp - jnp.max(sp, axis=-1, keepdims=True)
    ep = jnp.exp(sp)
    p = ep / jnp.sum(ep, axis=-1, keepdims=True)
    p = jnp.clip(p, 1e-10, 1.0 - 1e-10)
    q = jnp.tanh(jnp.einsum('bkd,dj->bkj', xc, wq_ref[...],
                            preferred_element_type=jnp.float32)[..., 0]
                 + bq_ref[0, 0])

    p_ref[...] = p
    q_ref[...] = q

    iota = lax.broadcasted_iota(jnp.int32, (B, K), 1)
    m = jnp.max(p, axis=-1, keepdims=True)
    am = jnp.min(jnp.where(p == m, iota, K), axis=-1)
    oh = (iota == am[:, None]).astype(jnp.float32)
    mc_ref[...] = jnp.einsum('bk,bkd->bd', oh, cand,
                             preferred_element_type=jnp.float32)
    mid_ref[...] = jnp.sum(oh * cid, axis=1, keepdims=True)


def _heads(tok_rows, img, candidates, tkb, did, wp, bp, wq, bq):
    return pl.pallas_call(
        _heads_body,
        out_shape=[
            jax.ShapeDtypeStruct((B, K), jnp.float32),
            jax.ShapeDtypeStruct((B, K), jnp.float32),
            jax.ShapeDtypeStruct((B, D), jnp.float32),
            jax.ShapeDtypeStruct((B, 1), jnp.float32),
            jax.ShapeDtypeStruct((B, K), jnp.float32),
        ],
    )(tok_rows, img, candidates, tkb, did, wp, bp, wq, bq)


def kernel(img, txt_tokens, txt_matric, actions_matric, data_emb, data_pre,
           data_id, tok_embed, W_seq0, b_seq0, W_p, b_p, W_q, b_q, k):
    del k
    b0 = b_seq0.reshape(1, 1).astype(jnp.float32)
    bp = b_p.reshape(1, 1).astype(jnp.float32)
    bq = b_q.reshape(1, 1).astype(jnp.float32)

    p_db, tk2d = _score_topk(txt_matric, actions_matric, W_seq0, b0)

    tokflat = txt_tokens.reshape(B * T * L).astype(jnp.int32)
    tok_idx = jnp.concatenate(
        [tokflat, jnp.zeros((TOKP - B * T * L,), jnp.int32)])
    tk512 = jnp.concatenate(
        [tk2d.reshape(B * K), jnp.zeros((CANDP - B * K,), jnp.int32)])
    tok_rows, cand_b = _sc_gather(tok_embed, tok_idx, data_emb, tk512)

    tkflat = tk2d.reshape(B * K)
    onehot = (tkflat[:, None] == jnp.arange(N, dtype=jnp.int32)[None, :]
              ).astype(jnp.bfloat16)
    pre_t = jnp.transpose(data_pre, (1, 2, 3, 0))
    pre_out = _pre_gather(onehot, pre_t)

    candidates = cand_b[:B * K].reshape(B, K, D)
    candidates_pre = pre_out.reshape(B, K, 3, 224, 224)

    p, q, max_candidates, mid, candidates_id = _heads(
        tok_rows, img, candidates, tk2d, data_id.reshape(1, N),
        W_p, bp, W_q, bq)
    max_candidates_id = mid.reshape(B)

    return (p, q, max_candidates, max_candidates_id, candidates,
            candidates_id, candidates_pre, p_db)
